# baseline (speedup 1.0000x reference)
import functools

import jax
import jax.numpy as jnp
from jax import lax
from jax.experimental import pallas as pl
from jax.experimental.pallas import tpu as pltpu
from jax.experimental.pallas import tpu_sc as plsc

N_NODES = 10000
N_EDGES = 320000
D = 128
D_OUT = 64
N_GRAPHS = 128

NC = 2
NS = 16
NW = NC * NS
EPT = N_EDGES // NW
CH = 128
IBLK = 8
NBLK = 10
EPT_PAD = NBLK * IBLK * CH
NPAD = 10240
ZR = NPAD // NS
WCH = 128


def _zero_fill(ref, nrows, ncols16, value=0.0):
    v16 = jnp.full((16,), value, jnp.float32)

    def row(i, _):
        for q in range(ncols16):
            ref[i, pl.ds(q * 16, 16)] = v16
        return 0

    lax.fori_loop(0, nrows, row, 0)


@functools.cache
def _sc_mesh():
    return plsc.VectorSubcoreMesh(core_axis_name="c", subcore_axis_name="s",
                                  num_cores=NC, num_subcores=NS)


_SC_PARAMS = pltpu.CompilerParams(use_tc_tiling_on_sc=False)


def _seg_sum_cnt_body(table, src_r, dst_r, agg_out, cnt_out,
                      srcv, dstv, rows, rows2, onesb, acc, accc,
                      sem0, sem1, semc):
    c = lax.axis_index("c")
    s = lax.axis_index("s")
    w = s * NC + c

    _zero_fill(rows, WCH, 8)
    _zero_fill(onesb, WCH, 1)
    base = s * ZR
    for q in range(ZR // WCH):
        pltpu.sync_copy(rows, acc.at[pl.ds(base + q * WCH, WCH)])
        pltpu.sync_copy(onesb, accc.at[pl.ds(base + q * WCH, WCH)])
    _zero_fill(onesb, CH, 1, value=1.0)
    plsc.subcore_barrier()

    def outer(b, _):
        pltpu.sync_copy(src_r.at[w, b], srcv)
        pltpu.sync_copy(dst_r.at[w, b], dstv)
        bufs = (rows, rows2)
        sems = (sem0, sem1)
        scat = [None] * IBLK
        cnts = [None] * IBLK
        for j in range(IBLK):
            if j >= 2:
                scat[j - 2].wait()
            pltpu.sync_copy(table.at[srcv.at[j]], bufs[j % 2])
            scat[j] = pltpu.async_copy(bufs[j % 2], acc.at[dstv.at[j]],
                                       sems[j % 2], add=True)
            cnts[j] = pltpu.async_copy(onesb, accc.at[dstv.at[j]], semc,
                                       add=True)
        scat[IBLK - 2].wait()
        scat[IBLK - 1].wait()
        for j in range(IBLK):
            cnts[j].wait()
        return 0

    lax.fori_loop(0, NBLK, outer, 0)
    plsc.subcore_barrier()

    for q in range(ZR // WCH):
        r0 = base + q * WCH
        pltpu.sync_copy(acc.at[pl.ds(r0, WCH)], rows)
        pltpu.sync_copy(rows, agg_out.at[c, pl.ds(r0, WCH)])
        pltpu.sync_copy(accc.at[pl.ds(r0, WCH)], onesb)
        pltpu.sync_copy(onesb, cnt_out.at[c, pl.ds(r0, WCH)])


@functools.cache
def _make_seg_sum_cnt_sc():
    return pl.kernel(
        _seg_sum_cnt_body,
        out_type=(
            jax.ShapeDtypeStruct((NC, NPAD, D), jnp.float32),
            jax.ShapeDtypeStruct((NC, NPAD, 16), jnp.float32),
        ),
        mesh=_sc_mesh(),
        compiler_params=_SC_PARAMS,
        scratch_types=[
            pltpu.VMEM((IBLK, CH), jnp.int32),
            pltpu.VMEM((IBLK, CH), jnp.int32),
            pltpu.VMEM((CH, D), jnp.float32),
            pltpu.VMEM((CH, D), jnp.float32),
            pltpu.VMEM((CH, 16), jnp.float32),
            pltpu.VMEM_SHARED((NPAD, D), jnp.float32),
            pltpu.VMEM_SHARED((NPAD, 16), jnp.float32),
            pltpu.SemaphoreType.DMA,
            pltpu.SemaphoreType.DMA,
            pltpu.SemaphoreType.DMA,
        ],
    )


def _seg_sum_cnt_sc(table, src_r, dst_r):
    return _make_seg_sum_cnt_sc()(table, src_r, dst_r)


def _seg_sum_body(table, src_r, dst_r, agg_out, srcv, dstv, rows, rows2, acc,
                  sem0, sem1):
    c = lax.axis_index("c")
    s = lax.axis_index("s")
    w = s * NC + c

    _zero_fill(rows, WCH, 8)
    base = s * ZR
    for q in range(ZR // WCH):
        pltpu.sync_copy(rows, acc.at[pl.ds(base + q * WCH, WCH)])
    plsc.subcore_barrier()

    def outer(b, _):
        pltpu.sync_copy(src_r.at[w, b], srcv)
        pltpu.sync_copy(dst_r.at[w, b], dstv)
        bufs = (rows, rows2)
        sems = (sem0, sem1)
        scat = [None] * IBLK
        for j in range(IBLK):
            if j >= 2:
                scat[j - 2].wait()
            pltpu.sync_copy(table.at[srcv.at[j]], bufs[j % 2])
            scat[j] = pltpu.async_copy(bufs[j % 2], acc.at[dstv.at[j]],
                                       sems[j % 2], add=True)
        scat[IBLK - 2].wait()
        scat[IBLK - 1].wait()
        return 0

    lax.fori_loop(0, NBLK, outer, 0)
    plsc.subcore_barrier()

    for q in range(ZR // WCH):
        r0 = base + q * WCH
        pltpu.sync_copy(acc.at[pl.ds(r0, WCH)], rows)
        pltpu.sync_copy(rows, agg_out.at[c, pl.ds(r0, WCH)])


@functools.cache
def _make_seg_sum_sc():
    return pl.kernel(
        _seg_sum_body,
        out_type=jax.ShapeDtypeStruct((NC, NPAD, D), jnp.float32),
        mesh=_sc_mesh(),
        compiler_params=_SC_PARAMS,
        scratch_types=[
            pltpu.VMEM((IBLK, CH), jnp.int32),
            pltpu.VMEM((IBLK, CH), jnp.int32),
            pltpu.VMEM((CH, D), jnp.float32),
            pltpu.VMEM((CH, D), jnp.float32),
            pltpu.VMEM_SHARED((NPAD, D), jnp.float32),
            pltpu.SemaphoreType.DMA,
            pltpu.SemaphoreType.DMA,
        ],
    )


def _seg_sum_sc(table, src_r, dst_r):
    return _make_seg_sum_sc()(table, src_r, dst_r)


_R = 2000
_G = N_NODES // _R


def _sage_layer_body(h_ref, a_ref, c_ref, wl_ref, wr_ref, bl_ref, g_ref,
                     be_ref, o_ref):
    agg = a_ref[0] + a_ref[1]
    cnt = c_ref[0][:, 0:1] + c_ref[1][:, 0:1]
    mean = agg / jnp.maximum(cnt, 1.0)
    z = (jnp.dot(mean, wl_ref[...], preferred_element_type=jnp.float32)
         + jnp.dot(h_ref[...], wr_ref[...], preferred_element_type=jnp.float32)
         + bl_ref[...])
    mu = jnp.mean(z, axis=1, keepdims=True)
    zc = z - mu
    var = jnp.mean(zc * zc, axis=1, keepdims=True)
    y = zc * lax.rsqrt(var + 1e-5) * g_ref[...] + be_ref[...]
    o_ref[...] = jnp.maximum(y, 0.0)


def _sage_layer_tc(h, agg2, cnt2, W_l, b_l, W_r, g, beta):
    return pl.pallas_call(
        _sage_layer_body,
        grid=(_G,),
        in_specs=[
            pl.BlockSpec((_R, D), lambda i: (i, 0)),
            pl.BlockSpec((NC, _R, D), lambda i: (0, i, 0)),
            pl.BlockSpec((NC, _R, 16), lambda i: (0, i, 0)),
            pl.BlockSpec((D, D), lambda i: (0, 0)),
            pl.BlockSpec((D, D), lambda i: (0, 0)),
            pl.BlockSpec((1, D), lambda i: (0, 0)),
            pl.BlockSpec((1, D), lambda i: (0, 0)),
            pl.BlockSpec((1, D), lambda i: (0, 0)),
        ],
        out_specs=pl.BlockSpec((_R, D), lambda i: (i, 0)),
        out_shape=jax.ShapeDtypeStruct((N_NODES, D), jnp.float32),
    )(h, agg2, cnt2, W_l, W_r, b_l.reshape(1, D), g.reshape(1, D),
      beta.reshape(1, D))


def _pool_body(h_ref, b_ref, wo_ref, bo_ref, o_ref, acc_ref, cg_ref):
    i = pl.program_id(0)

    @pl.when(i == 0)
    def _init():
        acc_ref[...] = jnp.zeros_like(acc_ref)
        cg_ref[...] = jnp.zeros_like(cg_ref)

    oneh = (b_ref[...] == lax.broadcasted_iota(jnp.int32, (_R, N_GRAPHS), 1)
            ).astype(jnp.float32)
    acc_ref[...] += lax.dot_general(oneh, h_ref[...], (((0,), (0,)), ((), ())),
                                    preferred_element_type=jnp.float32)
    cg_ref[...] += lax.dot_general(oneh, jnp.ones((_R, 1), jnp.float32),
                                   (((0,), (0,)), ((), ())),
                                   preferred_element_type=jnp.float32)

    @pl.when(i == _G - 1)
    def _fin():
        pooled = acc_ref[...] / jnp.maximum(cg_ref[...], 1.0)
        o_ref[...] = (jnp.dot(pooled, wo_ref[...],
                              preferred_element_type=jnp.float32) + bo_ref[...])


def _pool_tc(h, batch2d, W_out, b_out):
    return pl.pallas_call(
        _pool_body,
        grid=(_G,),
        in_specs=[
            pl.BlockSpec((_R, D), lambda i: (i, 0)),
            pl.BlockSpec((_R, 1), lambda i: (i, 0)),
            pl.BlockSpec((D, D_OUT), lambda i: (0, 0)),
            pl.BlockSpec((1, D_OUT), lambda i: (0, 0)),
        ],
        out_specs=pl.BlockSpec((N_GRAPHS, D_OUT), lambda i: (0, 0)),
        out_shape=jax.ShapeDtypeStruct((N_GRAPHS, D_OUT), jnp.float32),
        scratch_shapes=[pltpu.VMEM((N_GRAPHS, D), jnp.float32),
                        pltpu.VMEM((N_GRAPHS, 1), jnp.float32)],
    )(h, batch2d, W_out, b_out.reshape(1, D_OUT))


def kernel(x, edge_index, batch, W_l0, b_l0, W_r0, g0, beta0,
           W_l1, b_l1, W_r1, g1, beta1, W_out, b_out):
    pad = EPT_PAD - EPT
    src = edge_index[0].astype(jnp.int32).reshape(NW, EPT)
    src = jnp.pad(src, ((0, 0), (0, pad))).reshape(NW, NBLK, IBLK, CH)
    dst = edge_index[1].astype(jnp.int32).reshape(NW, EPT)
    dst = jnp.pad(dst, ((0, 0), (0, pad)),
                  constant_values=N_NODES).reshape(NW, NBLK, IBLK, CH)
    batch2d = batch.astype(jnp.int32).reshape(N_NODES, 1)

    agg0, cnt2 = _seg_sum_cnt_sc(x, src, dst)
    h1 = _sage_layer_tc(x, agg0, cnt2, W_l0, b_l0, W_r0, g0, beta0)
    agg1 = _seg_sum_sc(h1, src, dst)
    h2 = _sage_layer_tc(h1, agg1, cnt2, W_l1, b_l1, W_r1, g1, beta1)
    return _pool_tc(h2, batch2d, W_out, b_out)


# speedup vs baseline: 2.0204x; 2.0204x over previous
import functools

import jax
import jax.numpy as jnp
from jax import lax
from jax.experimental import pallas as pl
from jax.experimental.pallas import tpu as pltpu
from jax.experimental.pallas import tpu_sc as plsc

N_NODES = 10000
N_EDGES = 320000
D = 128
DH = 64
D_OUT = 64
N_GRAPHS = 128

NC = 2
NS = 16
EPT = N_EDGES // NS
CH = 128
IBLK = 8
NBLK = 20
EPT_PAD = NBLK * IBLK * CH
NPAD = 10240
ZR = NPAD // NS
SRT = N_NODES // NS
SCH = 125
NBUF = 3


def _zero_fill(ref, nrows, ncols16, value=0.0):
    v16 = jnp.full((16,), value, jnp.float32)

    def row(i, _):
        for q in range(ncols16):
            ref[i, pl.ds(q * 16, 16)] = v16
        return 0

    lax.fori_loop(0, nrows, row, 0)


@functools.cache
def _sc_mesh():
    return plsc.VectorSubcoreMesh(core_axis_name="c", subcore_axis_name="s",
                                  num_cores=NC, num_subcores=NS)


_SC_PARAMS = pltpu.CompilerParams(use_tc_tiling_on_sc=False)


def _stage_and_zero(table, tab, acc, rows, c, s):
    _zero_fill(rows, CH, DH // 16)
    base = s * ZR
    for q in range(ZR // CH):
        pltpu.sync_copy(rows, acc.at[pl.ds(base + q * CH, CH)])
    tbase = s * SRT
    for q in range(SRT // SCH):
        r0 = tbase + q * SCH
        pltpu.sync_copy(table.at[pl.ds(r0, SCH), pl.ds(c * DH, DH)],
                        rows.at[pl.ds(0, SCH)])
        pltpu.sync_copy(rows.at[pl.ds(0, SCH)], tab.at[pl.ds(r0, SCH)])


def _write_back(acc, rows, out, c, s):
    base = s * ZR
    for q in range(ZR // CH):
        r0 = base + q * CH
        pltpu.sync_copy(acc.at[pl.ds(r0, CH)], rows)
        pltpu.sync_copy(rows, out.at[c, pl.ds(r0, CH)])


def _seg_sum_cnt_body(table, src_r, dst_r, agg_out, cnt_out,
                      srcv, dstv, rows, rows2, rows3, onesb, tab, acc, accc,
                      sem0, sem1, sem2):
    c = lax.axis_index("c")
    s = lax.axis_index("s")

    _stage_and_zero(table, tab, acc, rows, c, s)
    _zero_fill(onesb, CH, 1)
    base = s * ZR
    for q in range(ZR // CH):
        pltpu.sync_copy(onesb, accc.at[pl.ds(base + q * CH, CH)])
    _zero_fill(onesb, CH, 1, value=1.0)
    plsc.subcore_barrier()

    bufs = (rows, rows2, rows3)
    sems = (sem0, sem1, sem2)

    def outer(b, _):
        pltpu.sync_copy(src_r.at[s, b], srcv)
        pltpu.sync_copy(dst_r.at[s, b], dstv)
        scat = [None] * IBLK
        for j in range(IBLK):
            if j >= NBUF:
                scat[j - NBUF].wait()
            pltpu.sync_copy(tab.at[srcv.at[j]], bufs[j % NBUF])
            scat[j] = pltpu.async_copy(bufs[j % NBUF], acc.at[dstv.at[j]],
                                       sems[j % NBUF], add=True)

        @pl.when((b % 2) == c)
        def _cnt():
            for j in range(IBLK):
                pltpu.sync_copy(onesb, accc.at[dstv.at[j]], add=True)

        for j in range(IBLK - NBUF, IBLK):
            scat[j].wait()
        return 0

    lax.fori_loop(0, NBLK, outer, 0)
    plsc.subcore_barrier()

    _write_back(acc, rows, agg_out, c, s)
    for q in range(ZR // CH):
        r0 = base + q * CH
        pltpu.sync_copy(accc.at[pl.ds(r0, CH)], onesb)
        pltpu.sync_copy(onesb, cnt_out.at[c, pl.ds(r0, CH)])


@functools.cache
def _make_seg_sum_cnt_sc():
    return pl.kernel(
        _seg_sum_cnt_body,
        out_type=(
            jax.ShapeDtypeStruct((NC, NPAD, DH), jnp.float32),
            jax.ShapeDtypeStruct((NC, NPAD, 16), jnp.float32),
        ),
        mesh=_sc_mesh(),
        compiler_params=_SC_PARAMS,
        scratch_types=[
            pltpu.VMEM((IBLK, CH), jnp.int32),
            pltpu.VMEM((IBLK, CH), jnp.int32),
            pltpu.VMEM((CH, DH), jnp.float32),
            pltpu.VMEM((CH, DH), jnp.float32),
            pltpu.VMEM((CH, DH), jnp.float32),
            pltpu.VMEM((CH, 16), jnp.float32),
            pltpu.VMEM_SHARED((NPAD, DH), jnp.float32),
            pltpu.VMEM_SHARED((NPAD, DH), jnp.float32),
            pltpu.VMEM_SHARED((NPAD, 16), jnp.float32),
            pltpu.SemaphoreType.DMA,
            pltpu.SemaphoreType.DMA,
            pltpu.SemaphoreType.DMA,
        ],
    )


def _seg_sum_cnt_sc(table, src_r, dst_r):
    return _make_seg_sum_cnt_sc()(table, src_r, dst_r)


def _seg_sum_body(table, src_r, dst_r, agg_out,
                  srcv, dstv, rows, rows2, rows3, tab, acc, sem0, sem1, sem2):
    c = lax.axis_index("c")
    s = lax.axis_index("s")

    _stage_and_zero(table, tab, acc, rows, c, s)
    plsc.subcore_barrier()

    bufs = (rows, rows2, rows3)
    sems = (sem0, sem1, sem2)

    def outer(b, _):
        pltpu.sync_copy(src_r.at[s, b], srcv)
        pltpu.sync_copy(dst_r.at[s, b], dstv)
        scat = [None] * IBLK
        for j in range(IBLK):
            if j >= NBUF:
                scat[j - NBUF].wait()
            pltpu.sync_copy(tab.at[srcv.at[j]], bufs[j % NBUF])
            scat[j] = pltpu.async_copy(bufs[j % NBUF], acc.at[dstv.at[j]],
                                       sems[j % NBUF], add=True)
        for j in range(IBLK - NBUF, IBLK):
            scat[j].wait()
        return 0

    lax.fori_loop(0, NBLK, outer, 0)
    plsc.subcore_barrier()

    _write_back(acc, rows, agg_out, c, s)


@functools.cache
def _make_seg_sum_sc():
    return pl.kernel(
        _seg_sum_body,
        out_type=jax.ShapeDtypeStruct((NC, NPAD, DH), jnp.float32),
        mesh=_sc_mesh(),
        compiler_params=_SC_PARAMS,
        scratch_types=[
            pltpu.VMEM((IBLK, CH), jnp.int32),
            pltpu.VMEM((IBLK, CH), jnp.int32),
            pltpu.VMEM((CH, DH), jnp.float32),
            pltpu.VMEM((CH, DH), jnp.float32),
            pltpu.VMEM((CH, DH), jnp.float32),
            pltpu.VMEM_SHARED((NPAD, DH), jnp.float32),
            pltpu.VMEM_SHARED((NPAD, DH), jnp.float32),
            pltpu.SemaphoreType.DMA,
            pltpu.SemaphoreType.DMA,
            pltpu.SemaphoreType.DMA,
        ],
    )


def _seg_sum_sc(table, src_r, dst_r):
    return _make_seg_sum_sc()(table, src_r, dst_r)


_R = 2000
_G = N_NODES // _R


def _sage_layer_body(h_ref, a_ref, c_ref, wl_ref, wr_ref, bl_ref, g_ref,
                     be_ref, o_ref):
    agg = jnp.concatenate([a_ref[0], a_ref[1]], axis=1)
    cnt = c_ref[0][:, 0:1] + c_ref[1][:, 0:1]
    mean = agg / jnp.maximum(cnt, 1.0)
    z = (jnp.dot(mean, wl_ref[...], preferred_element_type=jnp.float32)
         + jnp.dot(h_ref[...], wr_ref[...], preferred_element_type=jnp.float32)
         + bl_ref[...])
    mu = jnp.mean(z, axis=1, keepdims=True)
    zc = z - mu
    var = jnp.mean(zc * zc, axis=1, keepdims=True)
    y = zc * lax.rsqrt(var + 1e-5) * g_ref[...] + be_ref[...]
    o_ref[...] = jnp.maximum(y, 0.0)


def _sage_layer_tc(h, agg2, cnt2, W_l, b_l, W_r, g, beta):
    return pl.pallas_call(
        _sage_layer_body,
        grid=(_G,),
        in_specs=[
            pl.BlockSpec((_R, D), lambda i: (i, 0)),
            pl.BlockSpec((NC, _R, DH), lambda i: (0, i, 0)),
            pl.BlockSpec((NC, _R, 16), lambda i: (0, i, 0)),
            pl.BlockSpec((D, D), lambda i: (0, 0)),
            pl.BlockSpec((D, D), lambda i: (0, 0)),
            pl.BlockSpec((1, D), lambda i: (0, 0)),
            pl.BlockSpec((1, D), lambda i: (0, 0)),
            pl.BlockSpec((1, D), lambda i: (0, 0)),
        ],
        out_specs=pl.BlockSpec((_R, D), lambda i: (i, 0)),
        out_shape=jax.ShapeDtypeStruct((N_NODES, D), jnp.float32),
    )(h, agg2, cnt2, W_l, W_r, b_l.reshape(1, D), g.reshape(1, D),
      beta.reshape(1, D))


def _pool_body(h_ref, b_ref, wo_ref, bo_ref, o_ref, acc_ref, cg_ref):
    i = pl.program_id(0)

    @pl.when(i == 0)
    def _init():
        acc_ref[...] = jnp.zeros_like(acc_ref)
        cg_ref[...] = jnp.zeros_like(cg_ref)

    oneh = (b_ref[...] == lax.broadcasted_iota(jnp.int32, (_R, N_GRAPHS), 1)
            ).astype(jnp.float32)
    acc_ref[...] += lax.dot_general(oneh, h_ref[...], (((0,), (0,)), ((), ())),
                                    preferred_element_type=jnp.float32)
    cg_ref[...] += lax.dot_general(oneh, jnp.ones((_R, 1), jnp.float32),
                                   (((0,), (0,)), ((), ())),
                                   preferred_element_type=jnp.float32)

    @pl.when(i == _G - 1)
    def _fin():
        pooled = acc_ref[...] / jnp.maximum(cg_ref[...], 1.0)
        o_ref[...] = (jnp.dot(pooled, wo_ref[...],
                              preferred_element_type=jnp.float32) + bo_ref[...])


def _pool_tc(h, batch2d, W_out, b_out):
    return pl.pallas_call(
        _pool_body,
        grid=(_G,),
        in_specs=[
            pl.BlockSpec((_R, D), lambda i: (i, 0)),
            pl.BlockSpec((_R, 1), lambda i: (i, 0)),
            pl.BlockSpec((D, D_OUT), lambda i: (0, 0)),
            pl.BlockSpec((1, D_OUT), lambda i: (0, 0)),
        ],
        out_specs=pl.BlockSpec((N_GRAPHS, D_OUT), lambda i: (0, 0)),
        out_shape=jax.ShapeDtypeStruct((N_GRAPHS, D_OUT), jnp.float32),
        scratch_shapes=[pltpu.VMEM((N_GRAPHS, D), jnp.float32),
                        pltpu.VMEM((N_GRAPHS, 1), jnp.float32)],
    )(h, batch2d, W_out, b_out.reshape(1, D_OUT))


def kernel(x, edge_index, batch, W_l0, b_l0, W_r0, g0, beta0,
           W_l1, b_l1, W_r1, g1, beta1, W_out, b_out):
    pad = EPT_PAD - EPT
    src = edge_index[0].astype(jnp.int32).reshape(NS, EPT)
    src = jnp.pad(src, ((0, 0), (0, pad))).reshape(NS, NBLK, IBLK, CH)
    dst = edge_index[1].astype(jnp.int32).reshape(NS, EPT)
    dst = jnp.pad(dst, ((0, 0), (0, pad)),
                  constant_values=N_NODES).reshape(NS, NBLK, IBLK, CH)
    batch2d = batch.astype(jnp.int32).reshape(N_NODES, 1)

    agg0, cnt2 = _seg_sum_cnt_sc(x, src, dst)
    h1 = _sage_layer_tc(x, agg0, cnt2, W_l0, b_l0, W_r0, g0, beta0)
    agg1 = _seg_sum_sc(h1, src, dst)
    h2 = _sage_layer_tc(h1, agg1, cnt2, W_l1, b_l1, W_r1, g1, beta1)
    return _pool_tc(h2, batch2d, W_out, b_out)


# speedup vs baseline: 2.0218x; 1.0007x over previous
import functools

import jax
import jax.numpy as jnp
from jax import lax
from jax.experimental import pallas as pl
from jax.experimental.pallas import tpu as pltpu
from jax.experimental.pallas import tpu_sc as plsc

N_NODES = 10000
N_EDGES = 320000
D = 128
DH = 64
D_OUT = 64
N_GRAPHS = 128

NC = 2
NS = 16
EPT = N_EDGES // NS
CH = 128
IBLK = 8
NBLK = 20
EPT_PAD = NBLK * IBLK * CH
NPAD = 10240
ZR = NPAD // NS
SRT = N_NODES // NS
SCH = 125
NBUF = 3


def _zero_fill(ref, nrows, ncols16, value=0.0):
    v16 = jnp.full((16,), value, jnp.float32)

    def row(i, _):
        for q in range(ncols16):
            ref[i, pl.ds(q * 16, 16)] = v16
        return 0

    lax.fori_loop(0, nrows, row, 0)


@functools.cache
def _sc_mesh():
    return plsc.VectorSubcoreMesh(core_axis_name="c", subcore_axis_name="s",
                                  num_cores=NC, num_subcores=NS)


_SC_PARAMS = pltpu.CompilerParams(use_tc_tiling_on_sc=False)


def _stage_and_zero(table, tab, acc, rows, c, s):
    _zero_fill(rows, CH, DH // 16)
    base = s * ZR
    for q in range(ZR // CH):
        pltpu.sync_copy(rows, acc.at[pl.ds(base + q * CH, CH)])
    tbase = s * SRT
    for q in range(SRT // SCH):
        r0 = tbase + q * SCH
        pltpu.sync_copy(table.at[pl.ds(r0, SCH), pl.ds(c * DH, DH)],
                        rows.at[pl.ds(0, SCH)])
        pltpu.sync_copy(rows.at[pl.ds(0, SCH)], tab.at[pl.ds(r0, SCH)])


def _write_back(acc, rows, out, c, s):
    base = s * ZR
    for q in range(ZR // CH):
        r0 = base + q * CH
        pltpu.sync_copy(acc.at[pl.ds(r0, CH)], rows)
        pltpu.sync_copy(rows, out.at[c, pl.ds(r0, CH)])


def _seg_sum_cnt_body(table, src_r, dst_r, agg_out, cnt_out,
                      srcv, dstv, rows, rows2, rows3, onesb, tab, acc, accc,
                      sem0, sem1, sem2, sg0, sg1, sg2):
    c = lax.axis_index("c")
    s = lax.axis_index("s")

    _stage_and_zero(table, tab, acc, rows, c, s)
    _zero_fill(onesb, CH, 1)
    base = s * ZR
    for q in range(ZR // CH):
        pltpu.sync_copy(onesb, accc.at[pl.ds(base + q * CH, CH)])
    _zero_fill(onesb, CH, 1, value=1.0)
    plsc.subcore_barrier()

    bufs = (rows, rows2, rows3)
    sems = (sem0, sem1, sem2)
    semg = (sg0, sg1, sg2)

    def outer(b, _):
        pltpu.sync_copy(src_r.at[s, b], srcv)
        pltpu.sync_copy(dst_r.at[s, b], dstv)
        scat = [None] * IBLK
        gath = [None] * IBLK
        gath[0] = pltpu.async_copy(tab.at[srcv.at[0]], bufs[0], semg[0])
        for j in range(IBLK):
            if j + 1 < IBLK:
                if j + 1 >= NBUF:
                    scat[j + 1 - NBUF].wait()
                gath[j + 1] = pltpu.async_copy(
                    tab.at[srcv.at[j + 1]], bufs[(j + 1) % NBUF],
                    semg[(j + 1) % NBUF])
            gath[j].wait()
            scat[j] = pltpu.async_copy(bufs[j % NBUF], acc.at[dstv.at[j]],
                                       sems[j % NBUF], add=True)

        @pl.when((b % 2) == c)
        def _cnt():
            for j in range(IBLK):
                pltpu.sync_copy(onesb, accc.at[dstv.at[j]], add=True)

        for j in range(IBLK - NBUF, IBLK):
            scat[j].wait()
        return 0

    lax.fori_loop(0, NBLK, outer, 0)
    plsc.subcore_barrier()

    _write_back(acc, rows, agg_out, c, s)
    for q in range(ZR // CH):
        r0 = base + q * CH
        pltpu.sync_copy(accc.at[pl.ds(r0, CH)], onesb)
        pltpu.sync_copy(onesb, cnt_out.at[c, pl.ds(r0, CH)])


@functools.cache
def _make_seg_sum_cnt_sc():
    return pl.kernel(
        _seg_sum_cnt_body,
        out_type=(
            jax.ShapeDtypeStruct((NC, NPAD, DH), jnp.float32),
            jax.ShapeDtypeStruct((NC, NPAD, 16), jnp.float32),
        ),
        mesh=_sc_mesh(),
        compiler_params=_SC_PARAMS,
        scratch_types=[
            pltpu.VMEM((IBLK, CH), jnp.int32),
            pltpu.VMEM((IBLK, CH), jnp.int32),
            pltpu.VMEM((CH, DH), jnp.float32),
            pltpu.VMEM((CH, DH), jnp.float32),
            pltpu.VMEM((CH, DH), jnp.float32),
            pltpu.VMEM((CH, 16), jnp.float32),
            pltpu.VMEM_SHARED((NPAD, DH), jnp.float32),
            pltpu.VMEM_SHARED((NPAD, DH), jnp.float32),
            pltpu.VMEM_SHARED((NPAD, 16), jnp.float32),
            pltpu.SemaphoreType.DMA,
            pltpu.SemaphoreType.DMA,
            pltpu.SemaphoreType.DMA,
            pltpu.SemaphoreType.DMA,
            pltpu.SemaphoreType.DMA,
            pltpu.SemaphoreType.DMA,
        ],
    )


def _seg_sum_cnt_sc(table, src_r, dst_r):
    return _make_seg_sum_cnt_sc()(table, src_r, dst_r)


def _seg_sum_body(table, src_r, dst_r, agg_out,
                  srcv, dstv, rows, rows2, rows3, tab, acc,
                  sem0, sem1, sem2, sg0, sg1, sg2):
    c = lax.axis_index("c")
    s = lax.axis_index("s")

    _stage_and_zero(table, tab, acc, rows, c, s)
    plsc.subcore_barrier()

    bufs = (rows, rows2, rows3)
    sems = (sem0, sem1, sem2)
    semg = (sg0, sg1, sg2)

    def outer(b, _):
        pltpu.sync_copy(src_r.at[s, b], srcv)
        pltpu.sync_copy(dst_r.at[s, b], dstv)
        scat = [None] * IBLK
        gath = [None] * IBLK
        gath[0] = pltpu.async_copy(tab.at[srcv.at[0]], bufs[0], semg[0])
        for j in range(IBLK):
            if j + 1 < IBLK:
                if j + 1 >= NBUF:
                    scat[j + 1 - NBUF].wait()
                gath[j + 1] = pltpu.async_copy(
                    tab.at[srcv.at[j + 1]], bufs[(j + 1) % NBUF],
                    semg[(j + 1) % NBUF])
            gath[j].wait()
            scat[j] = pltpu.async_copy(bufs[j % NBUF], acc.at[dstv.at[j]],
                                       sems[j % NBUF], add=True)
        for j in range(IBLK - NBUF, IBLK):
            scat[j].wait()
        return 0

    lax.fori_loop(0, NBLK, outer, 0)
    plsc.subcore_barrier()

    _write_back(acc, rows, agg_out, c, s)


@functools.cache
def _make_seg_sum_sc():
    return pl.kernel(
        _seg_sum_body,
        out_type=jax.ShapeDtypeStruct((NC, NPAD, DH), jnp.float32),
        mesh=_sc_mesh(),
        compiler_params=_SC_PARAMS,
        scratch_types=[
            pltpu.VMEM((IBLK, CH), jnp.int32),
            pltpu.VMEM((IBLK, CH), jnp.int32),
            pltpu.VMEM((CH, DH), jnp.float32),
            pltpu.VMEM((CH, DH), jnp.float32),
            pltpu.VMEM((CH, DH), jnp.float32),
            pltpu.VMEM_SHARED((NPAD, DH), jnp.float32),
            pltpu.VMEM_SHARED((NPAD, DH), jnp.float32),
            pltpu.SemaphoreType.DMA,
            pltpu.SemaphoreType.DMA,
            pltpu.SemaphoreType.DMA,
            pltpu.SemaphoreType.DMA,
            pltpu.SemaphoreType.DMA,
            pltpu.SemaphoreType.DMA,
        ],
    )


def _seg_sum_sc(table, src_r, dst_r):
    return _make_seg_sum_sc()(table, src_r, dst_r)


_R = 2000
_G = N_NODES // _R


def _sage_layer_body(h_ref, a_ref, c_ref, wl_ref, wr_ref, bl_ref, g_ref,
                     be_ref, o_ref):
    agg = jnp.concatenate([a_ref[0], a_ref[1]], axis=1)
    cnt = c_ref[0][:, 0:1] + c_ref[1][:, 0:1]
    mean = agg / jnp.maximum(cnt, 1.0)
    z = (jnp.dot(mean, wl_ref[...], preferred_element_type=jnp.float32)
         + jnp.dot(h_ref[...], wr_ref[...], preferred_element_type=jnp.float32)
         + bl_ref[...])
    mu = jnp.mean(z, axis=1, keepdims=True)
    zc = z - mu
    var = jnp.mean(zc * zc, axis=1, keepdims=True)
    y = zc * lax.rsqrt(var + 1e-5) * g_ref[...] + be_ref[...]
    o_ref[...] = jnp.maximum(y, 0.0)


def _sage_layer_tc(h, agg2, cnt2, W_l, b_l, W_r, g, beta):
    return pl.pallas_call(
        _sage_layer_body,
        grid=(_G,),
        in_specs=[
            pl.BlockSpec((_R, D), lambda i: (i, 0)),
            pl.BlockSpec((NC, _R, DH), lambda i: (0, i, 0)),
            pl.BlockSpec((NC, _R, 16), lambda i: (0, i, 0)),
            pl.BlockSpec((D, D), lambda i: (0, 0)),
            pl.BlockSpec((D, D), lambda i: (0, 0)),
            pl.BlockSpec((1, D), lambda i: (0, 0)),
            pl.BlockSpec((1, D), lambda i: (0, 0)),
            pl.BlockSpec((1, D), lambda i: (0, 0)),
        ],
        out_specs=pl.BlockSpec((_R, D), lambda i: (i, 0)),
        out_shape=jax.ShapeDtypeStruct((N_NODES, D), jnp.float32),
    )(h, agg2, cnt2, W_l, W_r, b_l.reshape(1, D), g.reshape(1, D),
      beta.reshape(1, D))


def _pool_body(h_ref, b_ref, wo_ref, bo_ref, o_ref, acc_ref, cg_ref):
    i = pl.program_id(0)

    @pl.when(i == 0)
    def _init():
        acc_ref[...] = jnp.zeros_like(acc_ref)
        cg_ref[...] = jnp.zeros_like(cg_ref)

    oneh = (b_ref[...] == lax.broadcasted_iota(jnp.int32, (_R, N_GRAPHS), 1)
            ).astype(jnp.float32)
    acc_ref[...] += lax.dot_general(oneh, h_ref[...], (((0,), (0,)), ((), ())),
                                    preferred_element_type=jnp.float32)
    cg_ref[...] += lax.dot_general(oneh, jnp.ones((_R, 1), jnp.float32),
                                   (((0,), (0,)), ((), ())),
                                   preferred_element_type=jnp.float32)

    @pl.when(i == _G - 1)
    def _fin():
        pooled = acc_ref[...] / jnp.maximum(cg_ref[...], 1.0)
        o_ref[...] = (jnp.dot(pooled, wo_ref[...],
                              preferred_element_type=jnp.float32) + bo_ref[...])


def _pool_tc(h, batch2d, W_out, b_out):
    return pl.pallas_call(
        _pool_body,
        grid=(_G,),
        in_specs=[
            pl.BlockSpec((_R, D), lambda i: (i, 0)),
            pl.BlockSpec((_R, 1), lambda i: (i, 0)),
            pl.BlockSpec((D, D_OUT), lambda i: (0, 0)),
            pl.BlockSpec((1, D_OUT), lambda i: (0, 0)),
        ],
        out_specs=pl.BlockSpec((N_GRAPHS, D_OUT), lambda i: (0, 0)),
        out_shape=jax.ShapeDtypeStruct((N_GRAPHS, D_OUT), jnp.float32),
        scratch_shapes=[pltpu.VMEM((N_GRAPHS, D), jnp.float32),
                        pltpu.VMEM((N_GRAPHS, 1), jnp.float32)],
    )(h, batch2d, W_out, b_out.reshape(1, D_OUT))


def kernel(x, edge_index, batch, W_l0, b_l0, W_r0, g0, beta0,
           W_l1, b_l1, W_r1, g1, beta1, W_out, b_out):
    pad = EPT_PAD - EPT
    src = edge_index[0].astype(jnp.int32).reshape(NS, EPT)
    src = jnp.pad(src, ((0, 0), (0, pad))).reshape(NS, NBLK, IBLK, CH)
    dst = edge_index[1].astype(jnp.int32).reshape(NS, EPT)
    dst = jnp.pad(dst, ((0, 0), (0, pad)),
                  constant_values=N_NODES).reshape(NS, NBLK, IBLK, CH)
    batch2d = batch.astype(jnp.int32).reshape(N_NODES, 1)

    agg0, cnt2 = _seg_sum_cnt_sc(x, src, dst)
    h1 = _sage_layer_tc(x, agg0, cnt2, W_l0, b_l0, W_r0, g0, beta0)
    agg1 = _seg_sum_sc(h1, src, dst)
    h2 = _sage_layer_tc(h1, agg1, cnt2, W_l1, b_l1, W_r1, g1, beta1)
    return _pool_tc(h2, batch2d, W_out, b_out)


# speedup vs baseline: 2.0442x; 1.0111x over previous
import functools

import jax
import jax.numpy as jnp
from jax import lax
from jax.experimental import pallas as pl
from jax.experimental.pallas import tpu as pltpu
from jax.experimental.pallas import tpu_sc as plsc

N_NODES = 10000
N_EDGES = 320000
D = 128
DH = 64
D_OUT = 64
N_GRAPHS = 128

NC = 2
NS = 16
EPT = N_EDGES // NS
CH = 128
IBLK = 8
NBLK = 20
EPT_PAD = NBLK * IBLK * CH
NPAD = 10240
ZR = NPAD // NS
SRT = N_NODES // NS
SCH = 125
NBUF = 3


def _zero_fill(ref, nrows, ncols16, value=0.0):
    v16 = jnp.full((16,), value, jnp.float32)

    def row(i, _):
        for q in range(ncols16):
            ref[i, pl.ds(q * 16, 16)] = v16
        return 0

    lax.fori_loop(0, nrows, row, 0)


@functools.cache
def _sc_mesh():
    return plsc.VectorSubcoreMesh(core_axis_name="c", subcore_axis_name="s",
                                  num_cores=NC, num_subcores=NS)


_SC_PARAMS = pltpu.CompilerParams(use_tc_tiling_on_sc=False)


def _stage_and_zero(table, tab, acc, rows, c, s):
    _zero_fill(rows, CH, DH // 16)
    base = s * ZR
    for q in range(ZR // CH):
        pltpu.sync_copy(rows, acc.at[pl.ds(base + q * CH, CH)])
    tbase = s * SRT
    for q in range(SRT // SCH):
        r0 = tbase + q * SCH
        pltpu.sync_copy(table.at[pl.ds(r0, SCH), pl.ds(c * DH, DH)],
                        rows.at[pl.ds(0, SCH)])
        pltpu.sync_copy(rows.at[pl.ds(0, SCH)], tab.at[pl.ds(r0, SCH)])


def _write_back(acc, rows, out, c, s):
    base = s * ZR
    for q in range(ZR // CH):
        r0 = base + q * CH
        pltpu.sync_copy(acc.at[pl.ds(r0, CH)], rows)
        pltpu.sync_copy(rows, out.at[c, pl.ds(r0, CH)])


def _seg_sum_cnt_body(table, src_r, dst_r, agg_out, cnt_out,
                      srcv, dstv, rows, rows2, rows3, onesb, tab, acc, accc,
                      sem0, sem1, sem2, sg0, sg1, sg2):
    c = lax.axis_index("c")
    s = lax.axis_index("s")

    _stage_and_zero(table, tab, acc, rows, c, s)
    _zero_fill(onesb, CH, 1)
    base = s * ZR
    for q in range(ZR // CH):
        pltpu.sync_copy(onesb, accc.at[pl.ds(base + q * CH, CH)])
    _zero_fill(onesb, CH, 1, value=1.0)
    plsc.subcore_barrier()

    bufs = (rows, rows2, rows3)
    sems = (sem0, sem1, sem2)
    semg = (sg0, sg1, sg2)

    def outer(b, _):
        pltpu.sync_copy(src_r.at[s, b], srcv)
        pltpu.sync_copy(dst_r.at[s, b], dstv)
        scat = [None] * IBLK
        gath = [None] * IBLK
        gath[0] = pltpu.async_copy(tab.at[srcv.at[0]], bufs[0], semg[0])
        for j in range(IBLK):
            if j + 1 < IBLK:
                if j + 1 >= NBUF:
                    scat[j + 1 - NBUF].wait()
                gath[j + 1] = pltpu.async_copy(
                    tab.at[srcv.at[j + 1]], bufs[(j + 1) % NBUF],
                    semg[(j + 1) % NBUF])
            gath[j].wait()
            scat[j] = pltpu.async_copy(bufs[j % NBUF], acc.at[dstv.at[j]],
                                       sems[j % NBUF], add=True)

        @pl.when((b % 2) == c)
        def _cnt():
            for j in range(IBLK):
                pltpu.sync_copy(onesb, accc.at[dstv.at[j]], add=True)

        for j in range(IBLK - NBUF, IBLK):
            scat[j].wait()
        return 0

    lax.fori_loop(0, NBLK, outer, 0)
    plsc.subcore_barrier()

    _write_back(acc, rows, agg_out, c, s)
    for q in range(ZR // CH):
        r0 = base + q * CH
        pltpu.sync_copy(accc.at[pl.ds(r0, CH)], onesb)
        pltpu.sync_copy(onesb, cnt_out.at[c, pl.ds(r0, CH)])


@functools.cache
def _make_seg_sum_cnt_sc():
    return pl.kernel(
        _seg_sum_cnt_body,
        out_type=(
            jax.ShapeDtypeStruct((NC, NPAD, DH), jnp.float32),
            jax.ShapeDtypeStruct((NC, NPAD, 16), jnp.float32),
        ),
        mesh=_sc_mesh(),
        compiler_params=_SC_PARAMS,
        scratch_types=[
            pltpu.VMEM((IBLK, CH), jnp.int32),
            pltpu.VMEM((IBLK, CH), jnp.int32),
            pltpu.VMEM((CH, DH), jnp.float32),
            pltpu.VMEM((CH, DH), jnp.float32),
            pltpu.VMEM((CH, DH), jnp.float32),
            pltpu.VMEM((CH, 16), jnp.float32),
            pltpu.VMEM_SHARED((NPAD, DH), jnp.float32),
            pltpu.VMEM_SHARED((NPAD, DH), jnp.float32),
            pltpu.VMEM_SHARED((NPAD, 16), jnp.float32),
            pltpu.SemaphoreType.DMA,
            pltpu.SemaphoreType.DMA,
            pltpu.SemaphoreType.DMA,
            pltpu.SemaphoreType.DMA,
            pltpu.SemaphoreType.DMA,
            pltpu.SemaphoreType.DMA,
        ],
    )


def _seg_sum_cnt_sc(table, src_r, dst_r):
    return _make_seg_sum_cnt_sc()(table, src_r, dst_r)


def _seg_sum_body(table, src_r, dst_r, agg_out,
                  srcv, dstv, rows, rows2, rows3, tab, acc,
                  sem0, sem1, sem2, sg0, sg1, sg2):
    c = lax.axis_index("c")
    s = lax.axis_index("s")

    _stage_and_zero(table, tab, acc, rows, c, s)
    plsc.subcore_barrier()

    bufs = (rows, rows2, rows3)
    sems = (sem0, sem1, sem2)
    semg = (sg0, sg1, sg2)

    def outer(b, _):
        pltpu.sync_copy(src_r.at[s, b], srcv)
        pltpu.sync_copy(dst_r.at[s, b], dstv)
        scat = [None] * IBLK
        gath = [None] * IBLK
        gath[0] = pltpu.async_copy(tab.at[srcv.at[0]], bufs[0], semg[0])
        for j in range(IBLK):
            if j + 1 < IBLK:
                if j + 1 >= NBUF:
                    scat[j + 1 - NBUF].wait()
                gath[j + 1] = pltpu.async_copy(
                    tab.at[srcv.at[j + 1]], bufs[(j + 1) % NBUF],
                    semg[(j + 1) % NBUF])
            gath[j].wait()
            scat[j] = pltpu.async_copy(bufs[j % NBUF], acc.at[dstv.at[j]],
                                       sems[j % NBUF], add=True)
        for j in range(IBLK - NBUF, IBLK):
            scat[j].wait()
        return 0

    lax.fori_loop(0, NBLK, outer, 0)
    plsc.subcore_barrier()

    _write_back(acc, rows, agg_out, c, s)


@functools.cache
def _make_seg_sum_sc():
    return pl.kernel(
        _seg_sum_body,
        out_type=jax.ShapeDtypeStruct((NC, NPAD, DH), jnp.float32),
        mesh=_sc_mesh(),
        compiler_params=_SC_PARAMS,
        scratch_types=[
            pltpu.VMEM((IBLK, CH), jnp.int32),
            pltpu.VMEM((IBLK, CH), jnp.int32),
            pltpu.VMEM((CH, DH), jnp.float32),
            pltpu.VMEM((CH, DH), jnp.float32),
            pltpu.VMEM((CH, DH), jnp.float32),
            pltpu.VMEM_SHARED((NPAD, DH), jnp.float32),
            pltpu.VMEM_SHARED((NPAD, DH), jnp.float32),
            pltpu.SemaphoreType.DMA,
            pltpu.SemaphoreType.DMA,
            pltpu.SemaphoreType.DMA,
            pltpu.SemaphoreType.DMA,
            pltpu.SemaphoreType.DMA,
            pltpu.SemaphoreType.DMA,
        ],
    )


def _seg_sum_sc(table, src_r, dst_r):
    return _make_seg_sum_sc()(table, src_r, dst_r)


_R = 2000
_G = N_NODES // _R


def _sage_layer_body(h_ref, a_ref, c_ref, wl_ref, wr_ref, bl_ref, g_ref,
                     be_ref, o_ref):
    agg = jnp.concatenate([a_ref[0], a_ref[1]], axis=1)
    cnt = c_ref[0][:, 0:1] + c_ref[1][:, 0:1]
    mean = agg / jnp.maximum(cnt, 1.0)
    z = (jnp.dot(mean, wl_ref[...], preferred_element_type=jnp.float32)
         + jnp.dot(h_ref[...], wr_ref[...], preferred_element_type=jnp.float32)
         + bl_ref[...])
    mu = jnp.mean(z, axis=1, keepdims=True)
    zc = z - mu
    var = jnp.mean(zc * zc, axis=1, keepdims=True)
    y = zc * lax.rsqrt(var + 1e-5) * g_ref[...] + be_ref[...]
    o_ref[...] = jnp.maximum(y, 0.0)


def _sage_layer_tc(h, agg2, cnt2, W_l, b_l, W_r, g, beta):
    return pl.pallas_call(
        _sage_layer_body,
        grid=(_G,),
        in_specs=[
            pl.BlockSpec((_R, D), lambda i: (i, 0)),
            pl.BlockSpec((NC, _R, DH), lambda i: (0, i, 0)),
            pl.BlockSpec((NC, _R, 16), lambda i: (0, i, 0)),
            pl.BlockSpec((D, D), lambda i: (0, 0)),
            pl.BlockSpec((D, D), lambda i: (0, 0)),
            pl.BlockSpec((1, D), lambda i: (0, 0)),
            pl.BlockSpec((1, D), lambda i: (0, 0)),
            pl.BlockSpec((1, D), lambda i: (0, 0)),
        ],
        out_specs=pl.BlockSpec((_R, D), lambda i: (i, 0)),
        out_shape=jax.ShapeDtypeStruct((N_NODES, D), jnp.float32),
    )(h, agg2, cnt2, W_l, W_r, b_l.reshape(1, D), g.reshape(1, D),
      beta.reshape(1, D))


def _sage_layer_pool_body(h_ref, a_ref, c_ref, wl_ref, wr_ref, bl_ref, g_ref,
                          be_ref, b_ref, wo_ref, bo_ref, o_ref, acc_ref,
                          cg_ref):
    i = pl.program_id(0)
    agg = jnp.concatenate([a_ref[0], a_ref[1]], axis=1)
    cnt = c_ref[0][:, 0:1] + c_ref[1][:, 0:1]
    mean = agg / jnp.maximum(cnt, 1.0)
    z = (jnp.dot(mean, wl_ref[...], preferred_element_type=jnp.float32)
         + jnp.dot(h_ref[...], wr_ref[...], preferred_element_type=jnp.float32)
         + bl_ref[...])
    mu = jnp.mean(z, axis=1, keepdims=True)
    zc = z - mu
    var = jnp.mean(zc * zc, axis=1, keepdims=True)
    y = zc * lax.rsqrt(var + 1e-5) * g_ref[...] + be_ref[...]
    y = jnp.maximum(y, 0.0)

    @pl.when(i == 0)
    def _init():
        acc_ref[...] = jnp.zeros_like(acc_ref)
        cg_ref[...] = jnp.zeros_like(cg_ref)

    oneh = (b_ref[...] == lax.broadcasted_iota(jnp.int32, (_R, N_GRAPHS), 1)
            ).astype(jnp.float32)
    acc_ref[...] += lax.dot_general(oneh, y, (((0,), (0,)), ((), ())),
                                    preferred_element_type=jnp.float32)
    cg_ref[...] += lax.dot_general(oneh, jnp.ones((_R, 1), jnp.float32),
                                   (((0,), (0,)), ((), ())),
                                   preferred_element_type=jnp.float32)

    @pl.when(i == _G - 1)
    def _fin():
        pooled = acc_ref[...] / jnp.maximum(cg_ref[...], 1.0)
        o_ref[...] = (jnp.dot(pooled, wo_ref[...],
                              preferred_element_type=jnp.float32) + bo_ref[...])


def _sage_layer_pool_tc(h, agg2, cnt2, W_l, b_l, W_r, g, beta, batch2d,
                        W_out, b_out):
    return pl.pallas_call(
        _sage_layer_pool_body,
        grid=(_G,),
        in_specs=[
            pl.BlockSpec((_R, D), lambda i: (i, 0)),
            pl.BlockSpec((NC, _R, DH), lambda i: (0, i, 0)),
            pl.BlockSpec((NC, _R, 16), lambda i: (0, i, 0)),
            pl.BlockSpec((D, D), lambda i: (0, 0)),
            pl.BlockSpec((D, D), lambda i: (0, 0)),
            pl.BlockSpec((1, D), lambda i: (0, 0)),
            pl.BlockSpec((1, D), lambda i: (0, 0)),
            pl.BlockSpec((1, D), lambda i: (0, 0)),
            pl.BlockSpec((_R, 1), lambda i: (i, 0)),
            pl.BlockSpec((D, D_OUT), lambda i: (0, 0)),
            pl.BlockSpec((1, D_OUT), lambda i: (0, 0)),
        ],
        out_specs=pl.BlockSpec((N_GRAPHS, D_OUT), lambda i: (0, 0)),
        out_shape=jax.ShapeDtypeStruct((N_GRAPHS, D_OUT), jnp.float32),
        scratch_shapes=[pltpu.VMEM((N_GRAPHS, D), jnp.float32),
                        pltpu.VMEM((N_GRAPHS, 1), jnp.float32)],
    )(h, agg2, cnt2, W_l, W_r, b_l.reshape(1, D), g.reshape(1, D),
      beta.reshape(1, D), batch2d, W_out, b_out.reshape(1, D_OUT))


def _pool_body(h_ref, b_ref, wo_ref, bo_ref, o_ref, acc_ref, cg_ref):
    i = pl.program_id(0)

    @pl.when(i == 0)
    def _init():
        acc_ref[...] = jnp.zeros_like(acc_ref)
        cg_ref[...] = jnp.zeros_like(cg_ref)

    oneh = (b_ref[...] == lax.broadcasted_iota(jnp.int32, (_R, N_GRAPHS), 1)
            ).astype(jnp.float32)
    acc_ref[...] += lax.dot_general(oneh, h_ref[...], (((0,), (0,)), ((), ())),
                                    preferred_element_type=jnp.float32)
    cg_ref[...] += lax.dot_general(oneh, jnp.ones((_R, 1), jnp.float32),
                                   (((0,), (0,)), ((), ())),
                                   preferred_element_type=jnp.float32)

    @pl.when(i == _G - 1)
    def _fin():
        pooled = acc_ref[...] / jnp.maximum(cg_ref[...], 1.0)
        o_ref[...] = (jnp.dot(pooled, wo_ref[...],
                              preferred_element_type=jnp.float32) + bo_ref[...])


def _pool_tc(h, batch2d, W_out, b_out):
    return pl.pallas_call(
        _pool_body,
        grid=(_G,),
        in_specs=[
            pl.BlockSpec((_R, D), lambda i: (i, 0)),
            pl.BlockSpec((_R, 1), lambda i: (i, 0)),
            pl.BlockSpec((D, D_OUT), lambda i: (0, 0)),
            pl.BlockSpec((1, D_OUT), lambda i: (0, 0)),
        ],
        out_specs=pl.BlockSpec((N_GRAPHS, D_OUT), lambda i: (0, 0)),
        out_shape=jax.ShapeDtypeStruct((N_GRAPHS, D_OUT), jnp.float32),
        scratch_shapes=[pltpu.VMEM((N_GRAPHS, D), jnp.float32),
                        pltpu.VMEM((N_GRAPHS, 1), jnp.float32)],
    )(h, batch2d, W_out, b_out.reshape(1, D_OUT))


def kernel(x, edge_index, batch, W_l0, b_l0, W_r0, g0, beta0,
           W_l1, b_l1, W_r1, g1, beta1, W_out, b_out):
    pad = EPT_PAD - EPT
    src = edge_index[0].astype(jnp.int32).reshape(NS, EPT)
    src = jnp.pad(src, ((0, 0), (0, pad))).reshape(NS, NBLK, IBLK, CH)
    dst = edge_index[1].astype(jnp.int32).reshape(NS, EPT)
    dst = jnp.pad(dst, ((0, 0), (0, pad)),
                  constant_values=N_NODES).reshape(NS, NBLK, IBLK, CH)
    batch2d = batch.astype(jnp.int32).reshape(N_NODES, 1)

    agg0, cnt2 = _seg_sum_cnt_sc(x, src, dst)
    h1 = _sage_layer_tc(x, agg0, cnt2, W_l0, b_l0, W_r0, g0, beta0)
    agg1 = _seg_sum_sc(h1, src, dst)
    return _sage_layer_pool_tc(h1, agg1, cnt2, W_l1, b_l1, W_r1, g1, beta1,
                               batch2d, W_out, b_out)


# speedup vs baseline: 2.0446x; 1.0002x over previous
import functools

import jax
import jax.numpy as jnp
from jax import lax
from jax.experimental import pallas as pl
from jax.experimental.pallas import tpu as pltpu
from jax.experimental.pallas import tpu_sc as plsc

N_NODES = 10000
N_EDGES = 320000
D = 128
DH = 64
D_OUT = 64
N_GRAPHS = 128

NC = 2
NS = 16
EPT = N_EDGES // NS
CH = 128
IBLK = 8
NBLK = 20
EPT_PAD = NBLK * IBLK * CH
NPAD = 10240
ZR = NPAD // NS
SRT = N_NODES // NS
SCH = 125
NBUF = 3


def _zero_fill(ref, nrows, ncols16, value=0.0):
    v16 = jnp.full((16,), value, jnp.float32)

    def row(i, _):
        for q in range(ncols16):
            ref[i, pl.ds(q * 16, 16)] = v16
        return 0

    lax.fori_loop(0, nrows, row, 0)


@functools.cache
def _sc_mesh():
    return plsc.VectorSubcoreMesh(core_axis_name="c", subcore_axis_name="s",
                                  num_cores=NC, num_subcores=NS)


_SC_PARAMS = pltpu.CompilerParams(use_tc_tiling_on_sc=False)


def _stage_and_zero(table, tab, acc, rows, c, s):
    _zero_fill(rows, CH, DH // 16)
    base = s * ZR
    for q in range(ZR // CH):
        pltpu.sync_copy(rows, acc.at[pl.ds(base + q * CH, CH)])
    tbase = s * SRT
    for q in range(SRT // SCH):
        r0 = tbase + q * SCH
        pltpu.sync_copy(table.at[pl.ds(r0, SCH), pl.ds(c * DH, DH)],
                        rows.at[pl.ds(0, SCH)])
        pltpu.sync_copy(rows.at[pl.ds(0, SCH)], tab.at[pl.ds(r0, SCH)])


def _write_back(acc, rows, out, c, s):
    base = s * ZR
    for q in range(ZR // CH):
        r0 = base + q * CH
        pltpu.sync_copy(acc.at[pl.ds(r0, CH)], rows)
        pltpu.sync_copy(rows, out.at[c, pl.ds(r0, CH)])


def _seg_sum_cnt_body(table, src_r, dst_r, agg_out, cnt_out,
                      srcv, dstv, rows, rows2, rows3, onesb, tab, acc, accc,
                      sem0, sem1, sem2, sg0, sg1, sg2):
    c = lax.axis_index("c")
    s = lax.axis_index("s")

    _stage_and_zero(table, tab, acc, rows, c, s)
    _zero_fill(onesb, CH, 1)
    base = s * ZR
    for q in range(ZR // CH):
        pltpu.sync_copy(onesb, accc.at[pl.ds(base + q * CH, CH)])
    _zero_fill(onesb, CH, 1, value=1.0)
    plsc.subcore_barrier()

    bufs = (rows, rows2, rows3)
    sems = (sem0, sem1, sem2)
    semg = (sg0, sg1, sg2)

    def outer(b, _):
        pltpu.sync_copy(src_r.at[s, b], srcv)
        pltpu.sync_copy(dst_r.at[s, b], dstv)
        scat = [None] * IBLK
        gath = [None] * IBLK
        gath[0] = pltpu.async_copy(tab.at[srcv.at[0]], bufs[0], semg[0])
        for j in range(IBLK):
            if j + 1 < IBLK:
                if j + 1 >= NBUF:
                    scat[j + 1 - NBUF].wait()
                gath[j + 1] = pltpu.async_copy(
                    tab.at[srcv.at[j + 1]], bufs[(j + 1) % NBUF],
                    semg[(j + 1) % NBUF])
            gath[j].wait()
            scat[j] = pltpu.async_copy(bufs[j % NBUF], acc.at[dstv.at[j]],
                                       sems[j % NBUF], add=True)

        @pl.when((b % 2) == c)
        def _cnt():
            for j in range(IBLK):
                pltpu.sync_copy(onesb, accc.at[dstv.at[j]], add=True)

        for j in range(IBLK - NBUF, IBLK):
            scat[j].wait()
        return 0

    lax.fori_loop(0, NBLK, outer, 0)
    plsc.subcore_barrier()

    _write_back(acc, rows, agg_out, c, s)
    for q in range(ZR // CH):
        r0 = base + q * CH
        pltpu.sync_copy(accc.at[pl.ds(r0, CH)], onesb)
        pltpu.sync_copy(onesb, cnt_out.at[c, pl.ds(r0, CH)])


@functools.cache
def _make_seg_sum_cnt_sc():
    return pl.kernel(
        _seg_sum_cnt_body,
        out_type=(
            jax.ShapeDtypeStruct((NC, NPAD, DH), jnp.float32),
            jax.ShapeDtypeStruct((NC, NPAD, 16), jnp.float32),
        ),
        mesh=_sc_mesh(),
        compiler_params=_SC_PARAMS,
        scratch_types=[
            pltpu.VMEM((IBLK, CH), jnp.int32),
            pltpu.VMEM((IBLK, CH), jnp.int32),
            pltpu.VMEM((CH, DH), jnp.float32),
            pltpu.VMEM((CH, DH), jnp.float32),
            pltpu.VMEM((CH, DH), jnp.float32),
            pltpu.VMEM((CH, 16), jnp.float32),
            pltpu.VMEM_SHARED((NPAD, DH), jnp.float32),
            pltpu.VMEM_SHARED((NPAD, DH), jnp.float32),
            pltpu.VMEM_SHARED((NPAD, 16), jnp.float32),
            pltpu.SemaphoreType.DMA,
            pltpu.SemaphoreType.DMA,
            pltpu.SemaphoreType.DMA,
            pltpu.SemaphoreType.DMA,
            pltpu.SemaphoreType.DMA,
            pltpu.SemaphoreType.DMA,
        ],
    )


def _seg_sum_cnt_sc(table, src_r, dst_r):
    return _make_seg_sum_cnt_sc()(table, src_r, dst_r)


def _seg_sum_body(table, src_r, dst_r, agg_out,
                  srcv, dstv, rows, rows2, rows3, tab, acc,
                  sem0, sem1, sem2, sg0, sg1, sg2):
    c = lax.axis_index("c")
    s = lax.axis_index("s")

    _stage_and_zero(table, tab, acc, rows, c, s)
    plsc.subcore_barrier()

    bufs = (rows, rows2, rows3)
    sems = (sem0, sem1, sem2)
    semg = (sg0, sg1, sg2)

    def outer(b, _):
        pltpu.sync_copy(src_r.at[s, b], srcv)
        pltpu.sync_copy(dst_r.at[s, b], dstv)
        scat = [None] * IBLK
        gath = [None] * IBLK
        gath[0] = pltpu.async_copy(tab.at[srcv.at[0]], bufs[0], semg[0])
        for j in range(IBLK):
            if j + 1 < IBLK:
                if j + 1 >= NBUF:
                    scat[j + 1 - NBUF].wait()
                gath[j + 1] = pltpu.async_copy(
                    tab.at[srcv.at[j + 1]], bufs[(j + 1) % NBUF],
                    semg[(j + 1) % NBUF])
            gath[j].wait()
            scat[j] = pltpu.async_copy(bufs[j % NBUF], acc.at[dstv.at[j]],
                                       sems[j % NBUF], add=True)
        for j in range(IBLK - NBUF, IBLK):
            scat[j].wait()
        return 0

    lax.fori_loop(0, NBLK, outer, 0)
    plsc.subcore_barrier()

    _write_back(acc, rows, agg_out, c, s)


@functools.cache
def _make_seg_sum_sc():
    return pl.kernel(
        _seg_sum_body,
        out_type=jax.ShapeDtypeStruct((NC, NPAD, DH), jnp.float32),
        mesh=_sc_mesh(),
        compiler_params=_SC_PARAMS,
        scratch_types=[
            pltpu.VMEM((IBLK, CH), jnp.int32),
            pltpu.VMEM((IBLK, CH), jnp.int32),
            pltpu.VMEM((CH, DH), jnp.float32),
            pltpu.VMEM((CH, DH), jnp.float32),
            pltpu.VMEM((CH, DH), jnp.float32),
            pltpu.VMEM_SHARED((NPAD, DH), jnp.float32),
            pltpu.VMEM_SHARED((NPAD, DH), jnp.float32),
            pltpu.SemaphoreType.DMA,
            pltpu.SemaphoreType.DMA,
            pltpu.SemaphoreType.DMA,
            pltpu.SemaphoreType.DMA,
            pltpu.SemaphoreType.DMA,
            pltpu.SemaphoreType.DMA,
        ],
    )


def _seg_sum_sc(table, src_r, dst_r):
    return _make_seg_sum_sc()(table, src_r, dst_r)


_R = 2000
_G = N_NODES // _R


def _sage_layer_body(h_ref, a_ref, c_ref, wl_ref, wr_ref, bl_ref, g_ref,
                     be_ref, o_ref):
    agg = jnp.concatenate([a_ref[0], a_ref[1]], axis=1)
    cnt = c_ref[0][:, 0:1] + c_ref[1][:, 0:1]
    mean = agg / jnp.maximum(cnt, 1.0)
    z = (jnp.dot(mean, wl_ref[...], preferred_element_type=jnp.float32)
         + jnp.dot(h_ref[...], wr_ref[...], preferred_element_type=jnp.float32)
         + bl_ref[...])
    mu = jnp.mean(z, axis=1, keepdims=True)
    zc = z - mu
    var = jnp.mean(zc * zc, axis=1, keepdims=True)
    y = zc * lax.rsqrt(var + 1e-5) * g_ref[...] + be_ref[...]
    o_ref[...] = jnp.maximum(y, 0.0)


def _sage_layer_tc(h, agg2, cnt2, W_l, b_l, W_r, g, beta):
    return pl.pallas_call(
        _sage_layer_body,
        grid=(_G,),
        in_specs=[
            pl.BlockSpec((_R, D), lambda i: (i, 0)),
            pl.BlockSpec((NC, _R, DH), lambda i: (0, i, 0)),
            pl.BlockSpec((NC, _R, 16), lambda i: (0, i, 0)),
            pl.BlockSpec((D, D), lambda i: (0, 0)),
            pl.BlockSpec((D, D), lambda i: (0, 0)),
            pl.BlockSpec((1, D), lambda i: (0, 0)),
            pl.BlockSpec((1, D), lambda i: (0, 0)),
            pl.BlockSpec((1, D), lambda i: (0, 0)),
        ],
        out_specs=pl.BlockSpec((_R, D), lambda i: (i, 0)),
        out_shape=jax.ShapeDtypeStruct((N_NODES, D), jnp.float32),
    )(h, agg2, cnt2, W_l, W_r, b_l.reshape(1, D), g.reshape(1, D),
      beta.reshape(1, D))


def _sage_layer_pool_body(h_ref, a_ref, c_ref, wl_ref, wr_ref, bl_ref, g_ref,
                          be_ref, b_ref, wo_ref, bo_ref, o_ref, acc_ref,
                          cg_ref):
    i = pl.program_id(0)
    agg = jnp.concatenate([a_ref[0], a_ref[1]], axis=1)
    cnt = c_ref[0][:, 0:1] + c_ref[1][:, 0:1]
    mean = agg / jnp.maximum(cnt, 1.0)
    z = (jnp.dot(mean, wl_ref[...], preferred_element_type=jnp.float32)
         + jnp.dot(h_ref[...], wr_ref[...], preferred_element_type=jnp.float32)
         + bl_ref[...])
    mu = jnp.mean(z, axis=1, keepdims=True)
    zc = z - mu
    var = jnp.mean(zc * zc, axis=1, keepdims=True)
    y = zc * lax.rsqrt(var + 1e-5) * g_ref[...] + be_ref[...]
    y = jnp.maximum(y, 0.0)

    @pl.when(i == 0)
    def _init():
        acc_ref[...] = jnp.zeros_like(acc_ref)
        cg_ref[...] = jnp.zeros_like(cg_ref)

    oneh = (b_ref[...] == lax.broadcasted_iota(jnp.int32, (_R, N_GRAPHS), 1)
            ).astype(jnp.float32)
    acc_ref[...] += lax.dot_general(oneh, y, (((0,), (0,)), ((), ())),
                                    preferred_element_type=jnp.float32)
    cg_ref[...] += lax.dot_general(oneh, jnp.ones((_R, 1), jnp.float32),
                                   (((0,), (0,)), ((), ())),
                                   preferred_element_type=jnp.float32)

    @pl.when(i == _G - 1)
    def _fin():
        pooled = acc_ref[...] / jnp.maximum(cg_ref[...], 1.0)
        o_ref[...] = (jnp.dot(pooled, wo_ref[...],
                              preferred_element_type=jnp.float32) + bo_ref[...])


def _sage_layer_pool_tc(h, agg2, cnt2, W_l, b_l, W_r, g, beta, batch2d,
                        W_out, b_out):
    return pl.pallas_call(
        _sage_layer_pool_body,
        grid=(_G,),
        in_specs=[
            pl.BlockSpec((_R, D), lambda i: (i, 0)),
            pl.BlockSpec((NC, _R, DH), lambda i: (0, i, 0)),
            pl.BlockSpec((NC, _R, 16), lambda i: (0, i, 0)),
            pl.BlockSpec((D, D), lambda i: (0, 0)),
            pl.BlockSpec((D, D), lambda i: (0, 0)),
            pl.BlockSpec((1, D), lambda i: (0, 0)),
            pl.BlockSpec((1, D), lambda i: (0, 0)),
            pl.BlockSpec((1, D), lambda i: (0, 0)),
            pl.BlockSpec((_R, 1), lambda i: (i, 0)),
            pl.BlockSpec((D, D_OUT), lambda i: (0, 0)),
            pl.BlockSpec((1, D_OUT), lambda i: (0, 0)),
        ],
        out_specs=pl.BlockSpec((N_GRAPHS, D_OUT), lambda i: (0, 0)),
        out_shape=jax.ShapeDtypeStruct((N_GRAPHS, D_OUT), jnp.float32),
        scratch_shapes=[pltpu.VMEM((N_GRAPHS, D), jnp.float32),
                        pltpu.VMEM((N_GRAPHS, 1), jnp.float32)],
    )(h, agg2, cnt2, W_l, W_r, b_l.reshape(1, D), g.reshape(1, D),
      beta.reshape(1, D), batch2d, W_out, b_out.reshape(1, D_OUT))


def kernel(x, edge_index, batch, W_l0, b_l0, W_r0, g0, beta0,
           W_l1, b_l1, W_r1, g1, beta1, W_out, b_out):
    pad = EPT_PAD - EPT
    src = edge_index[0].astype(jnp.int32).reshape(NS, EPT)
    src = jnp.pad(src, ((0, 0), (0, pad))).reshape(NS, NBLK, IBLK, CH)
    dst = edge_index[1].astype(jnp.int32).reshape(NS, EPT)
    dst = jnp.pad(dst, ((0, 0), (0, pad)),
                  constant_values=N_NODES).reshape(NS, NBLK, IBLK, CH)
    batch2d = batch.astype(jnp.int32).reshape(N_NODES, 1)

    agg0, cnt2 = _seg_sum_cnt_sc(x, src, dst)
    h1 = _sage_layer_tc(x, agg0, cnt2, W_l0, b_l0, W_r0, g0, beta0)
    agg1 = _seg_sum_sc(h1, src, dst)
    return _sage_layer_pool_tc(h1, agg1, cnt2, W_l1, b_l1, W_r1, g1, beta1,
                               batch2d, W_out, b_out)
